# baseline (speedup 1.0000x reference)
import functools

import jax
import jax.numpy as jnp
from jax import lax
from jax.experimental import pallas as pl
from jax.experimental.pallas import tpu as pltpu
from jax.experimental.pallas import tpu_sc as plsc

N = 10000
E = 320000
D = 128
NC = 2
NS = 16
NW = NC * NS
K = 64
SEG = 80
NSEG = 4
CHD = 160
EPAD = NSEG * NS * SEG * K
TOTCH = EPAD // K
NPAD = NS * 640

_MESH = dict(core_axis_name="c", subcore_axis_name="s")


@functools.partial(
    pl.kernel,
    out_type=jax.ShapeDtypeStruct((NC, NPAD), jnp.float32),
    mesh=plsc.VectorSubcoreMesh(**_MESH),
    scratch_types=[
        pltpu.VMEM((CHD, K), jnp.int32),
        pltpu.VMEM((K,), jnp.float32),
        pltpu.VMEM((640,), jnp.float32),
        pltpu.VMEM_SHARED((NPAD,), jnp.float32),
    ],
)
def _deg_sc(dst_hbm, out_hbm, dst_v, ones_v, zb_v, acc):
    cid = lax.axis_index("c")
    sid = lax.axis_index("s")
    wid = sid * NC + cid

    for i in range(K // 16):
        ones_v[pl.ds(i * 16, 16)] = jnp.ones((16,), jnp.float32)

    def zf(i, _):
        zb_v[pl.ds(i * 16, 16)] = jnp.zeros((16,), jnp.float32)
        return ()
    lax.fori_loop(0, 40, zf, ())
    pltpu.sync_copy(zb_v, acc.at[pl.ds(sid * 640, 640)])

    pltpu.sync_copy(dst_hbm.at[wid], dst_v)
    plsc.subcore_barrier()

    def body(j, _):
        pltpu.sync_copy(ones_v, acc.at[dst_v.at[j]], add=True)
        return ()
    lax.fori_loop(0, CHD, body, ())

    plsc.subcore_barrier()
    pltpu.sync_copy(acc.at[pl.ds(sid * 640, 640)],
                    out_hbm.at[cid, pl.ds(sid * 640, 640)])


@functools.partial(
    pl.kernel,
    out_type=jax.ShapeDtypeStruct((NC, NPAD, D), jnp.float32),
    mesh=plsc.VectorSubcoreMesh(**_MESH),
    scratch_types=[
        pltpu.VMEM((SEG // 2, K), jnp.int32),
        pltpu.VMEM((SEG // 2, K), jnp.int32),
        pltpu.VMEM((4, K, D), jnp.float32),
        pltpu.VMEM((8, D), jnp.float32),
        pltpu.VMEM_SHARED((NPAD, D), jnp.float32),
        pltpu.SemaphoreType.DMA,
        pltpu.SemaphoreType.DMA,
        pltpu.SemaphoreType.DMA,
        pltpu.SemaphoreType.DMA,
    ],
)
def _msg_sc(h2_hbm, src_hbm, dst_hbm, out_hbm,
            src_v, dst_v, bufs, zrow_v, acc, sem0, sem1, sem2, sem3):
    cid = lax.axis_index("c")
    sid = lax.axis_index("s")
    sems = (sem0, sem1, sem2, sem3)
    STG = SEG // 2

    def run_stage(base):
        pltpu.sync_copy(src_hbm.at[pl.ds(base, STG)], src_v)
        pltpu.sync_copy(dst_hbm.at[pl.ds(base, STG)], dst_v)
        for b in range(4):
            pltpu.async_copy(h2_hbm.at[src_v.at[b]], bufs.at[b], sems[b])

        def body(t, _):
            j0 = 4 * t
            for b in range(4):
                j = j0 + b
                pltpu.make_async_copy(
                    h2_hbm.at[src_v.at[j]], bufs.at[b], sems[b]).wait()
                pltpu.sync_copy(bufs.at[b], acc.at[dst_v.at[j]], add=True)

                @pl.when(j + 4 < STG)
                def _():
                    pltpu.async_copy(
                        h2_hbm.at[src_v.at[j + 4]], bufs.at[b], sems[b])
            return ()
        lax.fori_loop(0, STG // 4, body, ())

    for r in range(8):
        for c in range(D // 16):
            zrow_v[r, pl.ds(c * 16, 16)] = jnp.zeros((16,), jnp.float32)

    def zero_acc(i, _):
        pltpu.sync_copy(zrow_v, acc.at[pl.ds(sid * 640 + i * 8, 8), :])
        return ()
    lax.fori_loop(0, 80, zero_acc, ())
    plsc.subcore_barrier()

    @pl.when(cid == 0)
    def _():
        for s in range(NSEG // 2):
            for h in range(2):
                run_stage((s * NS + sid) * SEG + h * STG)

    @pl.when(cid == 1)
    def _():
        for s in range(NSEG // 2, NSEG):
            for h in range(2):
                run_stage((s * NS + sid) * SEG + h * STG)

    plsc.subcore_barrier()
    pltpu.sync_copy(acc.at[pl.ds(sid * 640, 640), :],
                    out_hbm.at[cid, pl.ds(sid * 640, 640), :])


def _mm_fn(x_ref, w_ref, d0_ref, d1_ref, h2_ref):
    deg = d0_ref[...] + d1_ref[...]
    dinv = jnp.where(deg > 0, lax.rsqrt(jnp.maximum(deg, 1e-12)), 0.0)
    h = jnp.dot(x_ref[...], w_ref[...], preferred_element_type=jnp.float32)
    h2_ref[...] = h * dinv


def _fin_fn(agg_ref, d0_ref, d1_ref, x_ref, b_ref, o_ref):
    deg = d0_ref[...] + d1_ref[...]
    dinv = jnp.where(deg > 0, lax.rsqrt(jnp.maximum(deg, 1e-12)), 0.0)
    agg = agg_ref[0] + agg_ref[1]
    o_ref[...] = jnp.maximum(agg * dinv + b_ref[...], 0.0) + x_ref[...]


MBLK = 1000


def kernel(x, edge_index, W, b):
    src = edge_index[0].astype(jnp.int32)
    dst = edge_index[1].astype(jnp.int32)
    srcp = jnp.concatenate([src, jnp.zeros((EPAD - E,), jnp.int32)])
    dstp = jnp.concatenate([dst, jnp.full((EPAD - E,), N, jnp.int32)])

    degp = _deg_sc(dstp.reshape(NW, CHD, K))
    d0 = degp[0, :N].reshape(N, 1)
    d1 = degp[1, :N].reshape(N, 1)
    dspec = pl.BlockSpec((MBLK, 1), lambda i: (i, 0))

    h2 = pl.pallas_call(
        _mm_fn,
        grid=(N // MBLK,),
        in_specs=[
            pl.BlockSpec((MBLK, D), lambda i: (i, 0)),
            pl.BlockSpec((D, D), lambda i: (0, 0)),
            dspec,
            dspec,
        ],
        out_specs=pl.BlockSpec((MBLK, D), lambda i: (i, 0)),
        out_shape=jax.ShapeDtypeStruct((N, D), jnp.float32),
    )(x, W, d0, d1)

    aggp = _msg_sc(h2, srcp.reshape(TOTCH, K),
                   dstp.reshape(TOTCH, K))

    out = pl.pallas_call(
        _fin_fn,
        grid=(N // MBLK,),
        in_specs=[
            pl.BlockSpec((NC, MBLK, D), lambda i: (0, i, 0)),
            dspec,
            dspec,
            pl.BlockSpec((MBLK, D), lambda i: (i, 0)),
            pl.BlockSpec((1, D), lambda i: (0, 0)),
        ],
        out_specs=pl.BlockSpec((MBLK, D), lambda i: (i, 0)),
        out_shape=jax.ShapeDtypeStruct((N, D), jnp.float32),
    )(aggp, d0, d1, x, b.reshape(1, D))
    return out


# speedup vs baseline: 1.0934x; 1.0934x over previous
import functools

import jax
import jax.numpy as jnp
from jax import lax
from jax.experimental import pallas as pl
from jax.experimental.pallas import tpu as pltpu
from jax.experimental.pallas import tpu_sc as plsc

N = 10000
E = 320000
D = 128
NC = 2
NS = 16
NW = NC * NS
K = 64
SEG = 80
NSEG = 4
CHD = 160
EPAD = NSEG * NS * SEG * K
TOTCH = EPAD // K
NPAD = NS * 640

_MESH = dict(core_axis_name="c", subcore_axis_name="s")


@functools.partial(
    pl.kernel,
    out_type=jax.ShapeDtypeStruct((NC, NPAD), jnp.float32),
    mesh=plsc.VectorSubcoreMesh(**_MESH),
    scratch_types=[
        pltpu.VMEM((CHD, K), jnp.int32),
        pltpu.VMEM((K,), jnp.float32),
        pltpu.VMEM((640,), jnp.float32),
        pltpu.VMEM_SHARED((NPAD,), jnp.float32),
    ],
)
def _deg_sc(dst_hbm, out_hbm, dst_v, ones_v, zb_v, acc):
    cid = lax.axis_index("c")
    sid = lax.axis_index("s")
    wid = sid * NC + cid

    for i in range(K // 16):
        ones_v[pl.ds(i * 16, 16)] = jnp.ones((16,), jnp.float32)

    def zf(i, _):
        zb_v[pl.ds(i * 16, 16)] = jnp.zeros((16,), jnp.float32)
        return ()
    lax.fori_loop(0, 40, zf, ())
    pltpu.sync_copy(zb_v, acc.at[pl.ds(sid * 640, 640)])

    pltpu.sync_copy(dst_hbm.at[wid], dst_v)
    plsc.subcore_barrier()

    def body(j, _):
        pltpu.sync_copy(ones_v, acc.at[dst_v.at[j]], add=True)
        return ()
    lax.fori_loop(0, CHD, body, ())

    plsc.subcore_barrier()
    pltpu.sync_copy(acc.at[pl.ds(sid * 640, 640)],
                    out_hbm.at[cid, pl.ds(sid * 640, 640)])


@functools.partial(
    pl.kernel,
    out_type=jax.ShapeDtypeStruct((NC, NPAD, D), jnp.float32),
    mesh=plsc.VectorSubcoreMesh(**_MESH),
    scratch_types=[
        pltpu.VMEM((SEG // 2, K), jnp.int32),
        pltpu.VMEM((SEG // 2, K), jnp.int32),
        pltpu.VMEM((4, K, D), jnp.float32),
        pltpu.VMEM((8, D), jnp.float32),
        pltpu.VMEM_SHARED((NPAD, D), jnp.float32),
        pltpu.SemaphoreType.DMA,
        pltpu.SemaphoreType.DMA,
        pltpu.SemaphoreType.DMA,
        pltpu.SemaphoreType.DMA,
    ],
)
def _msg_sc(h2_hbm, src_hbm, dst_hbm, out_hbm,
            src_v, dst_v, bufs, zrow_v, acc, sem0, sem1, sem2, sem3):
    cid = lax.axis_index("c")
    sid = lax.axis_index("s")
    sems = (sem0, sem1, sem2, sem3)
    STG = SEG // 2

    def run_stage(base):
        pltpu.sync_copy(src_hbm.at[pl.ds(base, STG)], src_v)
        pltpu.sync_copy(dst_hbm.at[pl.ds(base, STG)], dst_v)
        for b in range(4):
            pltpu.async_copy(h2_hbm.at[src_v.at[b]], bufs.at[b], sems[b])

        def body(t, _):
            j0 = 4 * t
            for b in range(4):
                j = j0 + b
                pltpu.make_async_copy(
                    h2_hbm.at[src_v.at[j]], bufs.at[b], sems[b]).wait()
                pltpu.sync_copy(bufs.at[b], acc.at[dst_v.at[j]], add=True)

                @pl.when(j + 4 < STG)
                def _():
                    pltpu.async_copy(
                        h2_hbm.at[src_v.at[j + 4]], bufs.at[b], sems[b])
            return ()
        lax.fori_loop(0, STG // 4, body, ())

    for r in range(8):
        for c in range(D // 16):
            zrow_v[r, pl.ds(c * 16, 16)] = jnp.zeros((16,), jnp.float32)

    def zero_acc(i, _):
        pltpu.sync_copy(zrow_v, acc.at[pl.ds(sid * 640 + i * 8, 8), :])
        return ()
    lax.fori_loop(0, 80, zero_acc, ())
    plsc.subcore_barrier()

    @pl.when(cid == 0)
    def _():
        for s in range(NSEG // 2):
            for h in range(2):
                run_stage((s * NS + sid) * SEG + h * STG)

    @pl.when(cid == 1)
    def _():
        for s in range(NSEG // 2, NSEG):
            for h in range(2):
                run_stage((s * NS + sid) * SEG + h * STG)

    plsc.subcore_barrier()
    pltpu.sync_copy(acc.at[pl.ds(sid * 640, 640), :],
                    out_hbm.at[cid, pl.ds(sid * 640, 640), :])


def _mm_fn(x_ref, w_ref, deg_ref, h2_ref):
    deg = deg_ref[:, 0] + deg_ref[:, 1]
    dinv = jnp.where(deg > 0, lax.rsqrt(jnp.maximum(deg, 1e-12)), 0.0)
    h = jnp.dot(x_ref[...], w_ref[...], preferred_element_type=jnp.float32)
    h2_ref[...] = h * dinv[:, None]


def _fin_fn(agg_ref, deg_ref, x_ref, b_ref, o_ref):
    deg = deg_ref[:, 0] + deg_ref[:, 1]
    dinv = jnp.where(deg > 0, lax.rsqrt(jnp.maximum(deg, 1e-12)), 0.0)
    agg = agg_ref[0] + agg_ref[1]
    o_ref[...] = jnp.maximum(agg * dinv[:, None] + b_ref[...], 0.0) + x_ref[...]


MBLK = 1000


def kernel(x, edge_index, W, b):
    src = edge_index[0].astype(jnp.int32)
    dst = edge_index[1].astype(jnp.int32)
    srcp = jnp.concatenate([src, jnp.zeros((EPAD - E,), jnp.int32)])
    dstp = jnp.concatenate([dst, jnp.full((EPAD - E,), N, jnp.int32)])

    degp = _deg_sc(dstp.reshape(NW, CHD, K))
    deg2 = jnp.transpose(degp[:, :N])

    h2 = pl.pallas_call(
        _mm_fn,
        grid=(N // MBLK,),
        in_specs=[
            pl.BlockSpec((MBLK, D), lambda i: (i, 0)),
            pl.BlockSpec((D, D), lambda i: (0, 0)),
            pl.BlockSpec((MBLK, NC), lambda i: (i, 0)),
        ],
        out_specs=pl.BlockSpec((MBLK, D), lambda i: (i, 0)),
        out_shape=jax.ShapeDtypeStruct((N, D), jnp.float32),
    )(x, W, deg2)

    aggp = _msg_sc(h2, srcp.reshape(TOTCH, K),
                   dstp.reshape(TOTCH, K))

    out = pl.pallas_call(
        _fin_fn,
        grid=(N // MBLK,),
        in_specs=[
            pl.BlockSpec((NC, MBLK, D), lambda i: (0, i, 0)),
            pl.BlockSpec((MBLK, NC), lambda i: (i, 0)),
            pl.BlockSpec((MBLK, D), lambda i: (i, 0)),
            pl.BlockSpec((1, D), lambda i: (0, 0)),
        ],
        out_specs=pl.BlockSpec((MBLK, D), lambda i: (i, 0)),
        out_shape=jax.ShapeDtypeStruct((N, D), jnp.float32),
    )(aggp, deg2, x, b.reshape(1, D))
    return out
